# Initial kernel scaffold; baseline (speedup 1.0000x reference)
import jax
import jax.numpy as jnp
from jax.experimental import pallas as pl


def kernel(node_feature, node_coord, batch, params):
    raise NotImplementedError("write your pallas kernel here")


# speedup vs baseline: 3.7865x; 3.7865x over previous
import functools

import jax
import jax.numpy as jnp
from jax import lax
from jax.experimental import pallas as pl
from jax.experimental.pallas import tpu as pltpu
from jax.experimental.pallas import tpu_sc as plsc

C = 128
H = 4
DH = C // H
K = 32
B = 32
R0 = 0.12
R1 = 0.24
INF = 3.0e38

_SC_NC = 2
_SC_NS = 16
_SC_NW = _SC_NC * _SC_NS


def _gather_rows(table, idx):
    V, D = table.shape
    E = idx.shape[0]
    assert D % 16 == 0, D
    assert E % (8 * _SC_NW) == 0, E
    b_per_w = E // _SC_NW
    ch = min(b_per_w, 128)
    n_ch = b_per_w // ch
    assert b_per_w % ch == 0

    mesh = plsc.VectorSubcoreMesh(core_axis_name="c", subcore_axis_name="s")

    @functools.partial(
        pl.kernel,
        mesh=mesh,
        out_type=jax.ShapeDtypeStruct((E, D), jnp.float32),
        scratch_types=[
            pltpu.VMEM((ch,), jnp.int32),
            pltpu.VMEM((ch, D), jnp.float32),
            pltpu.SemaphoreType.DMA,
        ],
    )
    def k(table_hbm, idx_hbm, out_hbm, idx_v, rows_v, sem):
        wid = lax.axis_index("s") * _SC_NC + lax.axis_index("c")
        base = wid * b_per_w

        def body(i, carry):
            off = base + i * ch
            pltpu.sync_copy(idx_hbm.at[pl.ds(off, ch)], idx_v)
            pltpu.async_copy(table_hbm.at[idx_v], rows_v, sem).wait()
            pltpu.sync_copy(rows_v, out_hbm.at[pl.ds(off, ch)])
            return carry

        lax.fori_loop(0, n_ch, body, 0)

    return k(table, idx)


def _fps(cx, cy, cz, n_sample):
    Ntot = cx.shape[1]
    BATCH = 8
    assert n_sample % BATCH == 0

    def body(cx_ref, cy_ref, cz_ref, idx_ref, dists):
        s = pl.program_id(0)
        lanes = lax.broadcasted_iota(jnp.int32, (1, Ntot), 1)
        xv = cx_ref[...]
        yv = cy_ref[...]
        zv = cz_ref[...]

        def one_iter():
            dv = dists[...]
            m = jnp.max(dv)
            nxt = jnp.min(jnp.where(dv >= m, lanes, Ntot))
            oh = lanes == nxt
            xn = jnp.sum(jnp.where(oh, xv, 0.0))
            yn = jnp.sum(jnp.where(oh, yv, 0.0))
            zn = jnp.sum(jnp.where(oh, zv, 0.0))
            dxv = xv - xn
            dyv = yv - yn
            dzv = zv - zn
            nd = (dxv * dxv + dyv * dyv) + dzv * dzv
            dists[...] = jnp.minimum(dv, nd)
            return nxt

        def store(nxts):
            cols = [jnp.broadcast_to(v, (1, 1)).astype(jnp.int32) for v in nxts]
            idx_ref[...] = jnp.concatenate(cols, axis=0).reshape(BATCH, 1, 1)

        @pl.when(s == 0)
        def _first():
            oh0 = lanes == 0
            x0 = jnp.sum(jnp.where(oh0, xv, 0.0))
            y0 = jnp.sum(jnp.where(oh0, yv, 0.0))
            z0 = jnp.sum(jnp.where(oh0, zv, 0.0))
            dx0 = xv - x0
            dy0 = yv - y0
            dz0 = zv - z0
            dists[...] = (dx0 * dx0 + dy0 * dy0) + dz0 * dz0
            nxts = [jnp.int32(0)]
            for _ in range(BATCH - 1):
                nxts.append(one_iter())
            store(nxts)

        @pl.when(s > 0)
        def _rest():
            nxts = [one_iter() for _ in range(BATCH)]
            store(nxts)

    out = pl.pallas_call(
        body,
        grid=(n_sample // BATCH,),
        in_specs=[
            pl.BlockSpec((1, Ntot), lambda i: (0, 0)),
            pl.BlockSpec((1, Ntot), lambda i: (0, 0)),
            pl.BlockSpec((1, Ntot), lambda i: (0, 0)),
        ],
        out_specs=pl.BlockSpec((BATCH, 1, 1), lambda i: (i, 0, 0)),
        out_shape=jax.ShapeDtypeStruct((n_sample, 1, 1), jnp.int32),
        scratch_shapes=[pltpu.VMEM((1, Ntot), jnp.float32)],
        compiler_params=pltpu.CompilerParams(
            dimension_semantics=("arbitrary",)
        ),
    )(cx, cy, cz)
    return out.reshape(n_sample)


def _knn(dstx, dsty, dstz, srcx, srcy, srcz):
    Nd = dstx.shape[0]
    Ns = srcx.shape[1]
    Rd = min(128, Nd)
    assert Nd % Rd == 0

    def body(dx_ref, dy_ref, dz_ref, sx_ref, sy_ref, sz_ref,
             idx_ref, dist_ref, sxx_ref, sxy_ref, sxz_ref, d2s):
        sxv = sx_ref[...]
        syv = sy_ref[...]
        szv = sz_ref[...]
        ddx = dx_ref[...] - sxv
        ddy = dy_ref[...] - syv
        ddz = dz_ref[...] - szv
        d2s[...] = (ddx * ddx + ddy * ddy) + ddz * ddz
        lanes = lax.broadcasted_iota(jnp.int32, (Rd, Ns), 1)
        kl = lax.broadcasted_iota(jnp.int32, (Rd, K), 1)
        acc_i = jnp.zeros((Rd, K), jnp.int32)
        acc_d = jnp.zeros((Rd, K), jnp.float32)
        acc_x = jnp.zeros((Rd, K), jnp.float32)
        acc_y = jnp.zeros((Rd, K), jnp.float32)
        acc_z = jnp.zeros((Rd, K), jnp.float32)
        for k in range(K):
            dv = d2s[...]
            m = jnp.min(dv, axis=1, keepdims=True)
            j = jnp.min(jnp.where(dv <= m, lanes, Ns), axis=1, keepdims=True)
            ohj = lanes == j
            acc_i = jnp.where(kl == k, j, acc_i)
            dk = jnp.sqrt(jnp.maximum(m, 1e-12))
            acc_d = jnp.where(kl == k, dk, acc_d)
            xn = jnp.sum(jnp.where(ohj, sxv, 0.0), axis=1, keepdims=True)
            yn = jnp.sum(jnp.where(ohj, syv, 0.0), axis=1, keepdims=True)
            zn = jnp.sum(jnp.where(ohj, szv, 0.0), axis=1, keepdims=True)
            acc_x = jnp.where(kl == k, xn, acc_x)
            acc_y = jnp.where(kl == k, yn, acc_y)
            acc_z = jnp.where(kl == k, zn, acc_z)
            d2s[...] = jnp.where(ohj, INF, dv)
        idx_ref[...] = acc_i
        dist_ref[...] = acc_d
        sxx_ref[...] = acc_x
        sxy_ref[...] = acc_y
        sxz_ref[...] = acc_z

    ospec = pl.BlockSpec((Rd, K), lambda i: (i, 0))
    oshape = jax.ShapeDtypeStruct((Nd, K), jnp.float32)
    return pl.pallas_call(
        body,
        grid=(Nd // Rd,),
        in_specs=[
            pl.BlockSpec((Rd, 1), lambda i: (i, 0)),
            pl.BlockSpec((Rd, 1), lambda i: (i, 0)),
            pl.BlockSpec((Rd, 1), lambda i: (i, 0)),
            pl.BlockSpec((1, Ns), lambda i: (0, 0)),
            pl.BlockSpec((1, Ns), lambda i: (0, 0)),
            pl.BlockSpec((1, Ns), lambda i: (0, 0)),
        ],
        out_specs=[ospec, ospec, ospec, ospec, ospec],
        out_shape=[
            jax.ShapeDtypeStruct((Nd, K), jnp.int32),
            oshape, oshape, oshape, oshape,
        ],
        scratch_shapes=[pltpu.VMEM((Rd, Ns), jnp.float32)],
    )(dstx, dsty, dstz, srcx, srcy, srcz)


def _mm(x, w):
    M = x.shape[0]
    Nc = w.shape[1]

    def body(x_ref, w_ref, o_ref):
        o_ref[...] = jnp.dot(x_ref[...], w_ref[...],
                             preferred_element_type=jnp.float32)

    return pl.pallas_call(
        body,
        out_shape=jax.ShapeDtypeStruct((M, Nc), jnp.float32),
    )(x, w)


def _edge(gth, sxx, sxy, sxz, dist, dstx, dsty, dstz, fdst,
          Wdst, Wfc1, Wfc2, Wsh8, Swa8, S2_8, Wo, bo2, W1, W2, radius):
    Nd = fdst.shape[0]
    Rt = min(64, Nd)
    assert Nd % Rt == 0
    Et = Rt * K
    cutoff = 0.99 * radius
    width = cutoff / B
    centers = jnp.linspace(0.0, cutoff, B).reshape(1, B)

    def body(g_ref, sxx_ref, sxy_ref, sxz_ref, d_ref, dx_ref, dy_ref,
             dz_ref, fd_ref, cen_ref, wdst_ref, wfc1_ref, wfc2_ref, wsh_ref,
             swa_ref, s2_ref, wo_ref, bo_ref, w1_ref, w2_ref, o_ref):
        hsrcg = g_ref[:, :C]
        hvg = g_ref[:, C:]
        fd = fd_ref[...]
        hdst = jnp.dot(fd, wdst_ref[...], preferred_element_type=jnp.float32)
        d = d_ref[...]
        d3 = d[:, :, None]
        cb = jnp.broadcast_to(cen_ref[...].reshape(1, 1, B), (Rt, K, B))
        garg = (d3 - cb) / width
        env = 0.5 * (jnp.cos(jnp.pi * jnp.clip(d / cutoff, 0.0, 1.0)) + 1.0)
        rbf = jnp.exp(-0.5 * (garg * garg)) * env[:, :, None]
        rbf2 = rbf.reshape(Et, B)
        es = jnp.dot(
            jax.nn.relu(jnp.dot(rbf2, wfc1_ref[...],
                                preferred_element_type=jnp.float32)),
            wfc2_ref[...], preferred_element_type=jnp.float32)
        hdst_e = jnp.broadcast_to(hdst[:, None, :], (Rt, K, C)).reshape(Et, C)
        msg = jax.nn.relu(hsrcg + hdst_e + es)
        den = d3 + 1e-9
        ux = (dx_ref[...] - sxx_ref[...])[:, :, None] / den
        uy = (dy_ref[...] - sxy_ref[...])[:, :, None] / den
        uz = (dz_ref[...] - sxz_ref[...])[:, :, None] / den
        l8 = lax.broadcasted_iota(jnp.int32, (Rt, K, 8), 2)
        sh8 = (jnp.where(l8 == 0, 1.0, 0.0)
               + jnp.where(l8 == 1, 1.0, 0.0) * ux
               + jnp.where(l8 == 2, 1.0, 0.0) * uy
               + jnp.where(l8 == 3, 1.0, 0.0) * uz)
        shW = jnp.dot(sh8.reshape(Et, 8), wsh_ref[...],
                      preferred_element_type=jnp.float32)
        msg2 = msg * shW
        mrelu = jax.nn.relu(msg2)
        logits = jnp.dot(mrelu, swa_ref[...],
                         preferred_element_type=jnp.float32)
        maskb = jnp.where(d <= radius, 0.0, -1e9)
        maskf = jnp.where(d <= radius, 1.0, 0.0)
        lg = logits.reshape(Rt, K, 8) + maskb[:, :, None]
        mx = jnp.max(lg, axis=1, keepdims=True)
        ex = jnp.exp(lg - mx)
        sm = ex[:, 0, :]
        for k in range(1, K):
            sm = sm + ex[:, k, :]
        alpha = ex / sm[:, None, :] * maskf[:, :, None]
        alpha_e = jnp.dot(alpha.reshape(Et, 8), s2_ref[...],
                          preferred_element_type=jnp.float32,
                          precision=jax.lax.Precision.HIGHEST)
        wsum3 = (alpha_e * hvg).reshape(Rt, K, C)
        agg = wsum3[:, 0, :]
        for k in range(1, K):
            agg = agg + wsum3[:, k, :]
        x = fd + jnp.dot(agg, wo_ref[...],
                         preferred_element_type=jnp.float32) + bo_ref[...]
        y = x + jnp.dot(
            jax.nn.relu(jnp.dot(x, w1_ref[...],
                                preferred_element_type=jnp.float32)),
            w2_ref[...], preferred_element_type=jnp.float32)
        o_ref[...] = y

    full = lambda shape: pl.BlockSpec(shape, lambda i: tuple(0 for _ in shape))
    kspec = pl.BlockSpec((Rt, K), lambda i: (i, 0))
    cspec = pl.BlockSpec((Rt, 1), lambda i: (i, 0))
    return pl.pallas_call(
        body,
        grid=(Nd // Rt,),
        in_specs=[
            pl.BlockSpec((Et, 2 * C), lambda i: (i, 0)),
            kspec, kspec, kspec, kspec,
            cspec, cspec, cspec,
            pl.BlockSpec((Rt, C), lambda i: (i, 0)),
            full((1, B)),
            full((C, C)),
            full((B, 32)),
            full((32, C)),
            full((8, C)),
            full((C, 8)),
            full((8, C)),
            full((C, C)),
            full((1, C)),
            full((C, 3 * C)),
            full((3 * C, C)),
        ],
        out_specs=pl.BlockSpec((Rt, C), lambda i: (i, 0)),
        out_shape=jax.ShapeDtypeStruct((Nd, C), jnp.float32),
    )(gth, sxx, sxy, sxz, dist, dstx, dsty, dstz, fdst, centers,
      Wdst, Wfc1, Wfc2, Wsh8, Swa8, S2_8, Wo, bo2, W1, W2)


def _prep(p, parity):
    Wcat = jnp.concatenate([p["Wmsg"], p["Wv"]], axis=1)
    sign = -1.0 if parity else 1.0
    Wsh8 = jnp.zeros((8, C), jnp.float32)
    Wsh8 = Wsh8.at[0].set(p["Wsh"][0])
    Wsh8 = Wsh8.at[1:4].set(sign * p["Wsh"][1:4])
    wa = p["wa"]
    Swa8 = jnp.zeros((C, 8), jnp.float32)
    S2_8 = jnp.zeros((8, C), jnp.float32)
    for h in range(H):
        Swa8 = Swa8.at[h * DH:(h + 1) * DH, h].set(wa[h])
        S2_8 = S2_8.at[h, h * DH:(h + 1) * DH].set(1.0)
    return Wcat, Wsh8, Swa8, S2_8


def _block(p, parity, graph, feat_src, feat_dst, radius):
    idxf, dist, sxx, sxy, sxz, ddx, ddy, ddz = graph
    Wcat, Wsh8, Swa8, S2_8 = _prep(p, parity)
    cat = _mm(feat_src, Wcat)
    gth = _gather_rows(cat, idxf)
    return _edge(gth, sxx, sxy, sxz, dist, ddx, ddy, ddz, feat_dst,
                 p["Wdst"], p["Wfc1"], p["Wfc2"], Wsh8, Swa8, S2_8,
                 p["Wo"], p["bo"][None, :], p["W1"], p["W2"], radius)


def _make_graph(dxyz, sxyz):
    ddx = dxyz[:, 0:1]
    ddy = dxyz[:, 1:2]
    ddz = dxyz[:, 2:3]
    ssx = sxyz[:, 0].reshape(1, -1)
    ssy = sxyz[:, 1].reshape(1, -1)
    ssz = sxyz[:, 2].reshape(1, -1)
    idx, dist, sxx, sxy, sxz = _knn(ddx, ddy, ddz, ssx, ssy, ssz)
    return idx.reshape(-1), dist, sxx, sxy, sxz, ddx, ddy, ddz


def kernel(node_feature, node_coord, batch, params):
    blocks = params["blocks"]
    N = node_coord.shape[0]
    n1 = N // 2
    n2 = n1 // 2

    c128_0 = jnp.zeros((N, C), jnp.float32).at[:, 0:3].set(node_coord)
    cx0 = node_coord[:, 0].reshape(1, N)
    cy0 = node_coord[:, 1].reshape(1, N)
    cz0 = node_coord[:, 2].reshape(1, N)

    idx1 = _fps(cx0, cy0, cz0, n1)
    c128_1 = _gather_rows(c128_0, idx1)
    xyz1 = c128_1[:, 0:3]
    cx1 = c128_1[:, 0].reshape(1, n1)
    cy1 = c128_1[:, 1].reshape(1, n1)
    cz1 = c128_1[:, 2].reshape(1, n1)

    idx2 = _fps(cx1, cy1, cz1, n2)
    c128_2 = _gather_rows(c128_1, idx2)
    xyz2 = c128_2[:, 0:3]

    gA = _make_graph(xyz1, node_coord)
    gB = _make_graph(xyz1, xyz1)
    gC = _make_graph(xyz2, xyz1)
    gD = _make_graph(xyz2, xyz2)
    gE = _make_graph(xyz1, xyz2)
    gF = _make_graph(node_coord, xyz1)

    f1d = _gather_rows(node_feature, idx1)
    f1 = _block(blocks[0], False, gA, node_feature, f1d, R0)
    f1 = _block(blocks[1], False, gB, f1, f1, R0)
    f2d = _gather_rows(f1, idx2)
    f2 = _block(blocks[2], False, gC, f1, f2d, R1)
    f2 = _block(blocks[3], False, gD, f2, f2, R1)
    f2 = _block(blocks[4], False, gD, f2, f2, R1)
    f2 = _block(blocks[5], False, gD, f2, f2, R1)
    f2 = _block(blocks[6], False, gD, f2, f2, R1)
    f1u = _block(blocks[7], True, gE, f2, f1, R1)
    f1u = _block(blocks[8], False, gB, f1u, f1u, R0)
    f0 = _block(blocks[9], True, gF, f1u, node_feature, R0)
    return f0
